# baseline (speedup 1.0000x reference)
import jax
import jax.numpy as jnp
from jax import lax
from jax.experimental import pallas as pl
from jax.experimental.pallas import tpu as pltpu
from jax.experimental.pallas import tpu_sc as plsc

D = 64
B = 16384
L = 16
NC, NS = 2, 16
NW = NC * NS
BPW = B // NW
R = 8
NT = 6
MARGIN = 1.0


def _rsqrt16(x):
    x = jnp.maximum(x, 1e-12)
    i = plsc.bitcast(x, jnp.int32)
    y = plsc.bitcast(jnp.full((L,), 0x5F3759DF, jnp.int32) - (i >> 1),
                     jnp.float32)
    for _ in range(3):
        y = y * (1.5 - 0.5 * x * y * y)
    return y


def _body(ph_i, pt_i, pr_i, nh_i, nt_i, nr_i, ent, rel,
          pred_out, loss_out,
          ph_x, pt_x, pr_x, nh_x, nt_x, nr_x,
          ring, pred_s, loss_s, sem):
    wid = lax.axis_index("s") * NC + lax.axis_index("c")
    base = wid * BPW
    row_iota = lax.iota(jnp.int32, L)
    lane0 = row_iota == 0
    zf = jnp.zeros((L,), jnp.float32)

    idx_refs = (ph_x, pt_x, pr_x, nh_x, nt_x, nr_x)
    idx_srcs = (ph_i, pt_i, pr_i, nh_i, nt_i, nr_i)
    tables = (ent, ent, rel, ent, ent, rel)
    NQ = D // L

    for src, dst in zip(idx_srcs, idx_refs):
        pltpu.sync_copy(src.at[pl.ds(base, BPW)], dst.at[pl.ds(0, BPW)])

    def fire(row):
        slot = row & (R - 1)
        for t, (tab, ix) in enumerate(zip(tables, idx_refs)):
            r0 = ix[pl.ds(row, L)][0]
            pltpu.async_copy(tab.at[pl.ds(r0, 1)],
                             ring.at[pl.ds(slot * NT + t, 1)], sem)

    for j in range(R):
        fire(j)

    def score_row(i):
        slot = i & (R - 1)
        quads = [[ring[slot * NT + t, pl.ds(L * q, L)] for q in range(NQ)]
                 for t in range(NT)]
        phq, ptq, prq, nhq, ntq, nrq = quads

        def inv_norm(vq):
            s = vq[0] * vq[0] + vq[1] * vq[1]
            s = s + vq[2] * vq[2] + vq[3] * vq[3]
            return _rsqrt16(jnp.full((L,), jnp.sum(s), jnp.float32))

        ih, it, ir, jh, jt, jr = [inv_norm(vq) for vq in quads]

        pa, na = zf, zf
        for q in range(NQ):
            pa = pa + jnp.abs(phq[q] * ih + prq[q] * ir - ptq[q] * it)
            na = na + jnp.abs(nhq[q] * jh + nrq[q] * jr - ntq[q] * jt)
        pv = jnp.full((L,), jnp.sum(pa), jnp.float32)
        nv = jnp.full((L,), jnp.sum(na), jnp.float32)
        plsc.store_scatter(pred_s, [jnp.full((L,), i, jnp.int32)],
                           pv, mask=lane0)
        return jnp.maximum(pv - nv + MARGIN, 0.0)

    def pair_body(i2, l_acc):
        i = i2 * 2
        for _ in range(2 * NT):
            pltpu.make_async_copy(ent.at[pl.ds(0, 1)],
                                  ring.at[pl.ds(0, 1)], sem).wait()
        rel0 = score_row(i)
        rel1 = score_row(i + 1)

        @pl.when(i2 < (BPW - R) // 2)
        def _():
            fire(i + R)
            fire(i + R + 1)

        return l_acc + rel0 + rel1

    loss_acc = lax.fori_loop(0, BPW // 2, pair_body, zf)

    loss_s[...] = loss_acc * 0.0625
    pltpu.sync_copy(pred_s, pred_out.at[pl.ds(base, BPW)])
    pltpu.sync_copy(loss_s, loss_out.at[wid])


def kernel(pos_h, pos_t, pos_r, neg_h, neg_t, neg_r,
           ent_embeddings, rel_embeddings):
    mesh = plsc.VectorSubcoreMesh(core_axis_name="c", subcore_axis_name="s")
    run = pl.kernel(
        _body,
        out_type=(
            jax.ShapeDtypeStruct((B,), jnp.float32),
            jax.ShapeDtypeStruct((NW, L), jnp.float32),
        ),
        mesh=mesh,
        compiler_params=pltpu.CompilerParams(needs_layout_passes=False,
                                             use_tc_tiling_on_sc=True),
        scratch_types=(
            [pltpu.VMEM((BPW + L,), jnp.int32) for _ in range(6)]
            + [pltpu.VMEM((R * NT, D), jnp.float32),
               pltpu.VMEM((BPW,), jnp.float32),
               pltpu.VMEM((L,), jnp.float32),
               pltpu.SemaphoreType.DMA]
        ),
    )
    pred, loss_part = run(
        pos_h.astype(jnp.int32), pos_t.astype(jnp.int32),
        pos_r.astype(jnp.int32), neg_h.astype(jnp.int32),
        neg_t.astype(jnp.int32), neg_r.astype(jnp.int32),
        ent_embeddings, rel_embeddings)
    return (jnp.sum(loss_part), pred)


# speedup vs baseline: 1.0666x; 1.0666x over previous
import jax
import jax.numpy as jnp
from jax import lax
from jax.experimental import pallas as pl
from jax.experimental.pallas import tpu as pltpu
from jax.experimental.pallas import tpu_sc as plsc

D = 64
B = 16384
L = 16
NC, NS = 2, 16
NW = NC * NS
BPW = B // NW
R = 8
NT = 6
MARGIN = 1.0


def _rsqrt16(x):
    x = jnp.maximum(x, 1e-12)
    i = plsc.bitcast(x, jnp.int32)
    y = plsc.bitcast(jnp.full((L,), 0x5F3759DF, jnp.int32) - (i >> 1),
                     jnp.float32)
    for _ in range(3):
        y = y * (1.5 - 0.5 * x * y * y)
    return y


def _body(ph_i, pt_i, pr_i, nh_i, nt_i, nr_i, ent, rel,
          pred_out, loss_out,
          ph_x, pt_x, pr_x, nh_x, nt_x, nr_x,
          ring, pred_s, loss_s, sem):
    wid = lax.axis_index("s") * NC + lax.axis_index("c")
    base = wid * BPW
    row_iota = lax.iota(jnp.int32, L)
    lane0 = row_iota == 0
    zf = jnp.zeros((L,), jnp.float32)

    idx_refs = (ph_x, pt_x, pr_x, nh_x, nt_x, nr_x)
    idx_srcs = (ph_i, pt_i, pr_i, nh_i, nt_i, nr_i)
    tables = (ent, ent, rel, ent, ent, rel)
    NQ = D // L

    for src, dst in zip(idx_srcs, idx_refs):
        pltpu.sync_copy(src.at[pl.ds(base, BPW)], dst.at[pl.ds(0, BPW)])

    def fire(row):
        slot = lax.rem(row, R)
        for t, (tab, ix) in enumerate(zip(tables, idx_refs)):
            r0 = ix[pl.ds(row, L)][0]
            pltpu.async_copy(tab.at[pl.ds(r0, 1)],
                             ring.at[pl.ds(slot * NT + t, 1)], sem)

    def drain_one():
        pltpu.make_async_copy(ent.at[pl.ds(0, 1)],
                              ring.at[pl.ds(0, 1)], sem).wait()

    for j in range(R):
        fire(j)

    def row_body(i, l_acc):
        slot = lax.rem(i, R)
        for _ in range(NT):
            drain_one()
        quads = [[ring[slot * NT + t, pl.ds(L * q, L)] for q in range(NQ)]
                 for t in range(NT)]

        @pl.when(i < BPW - R)
        def _():
            fire(i + R)

        phq, ptq, prq, nhq, ntq, nrq = quads

        def inv_norm(vq):
            s = vq[0] * vq[0] + vq[1] * vq[1]
            s = s + vq[2] * vq[2] + vq[3] * vq[3]
            return _rsqrt16(jnp.full((L,), jnp.sum(s), jnp.float32))

        ih, it, ir, jh, jt, jr = [inv_norm(vq) for vq in quads]

        pa, na = zf, zf
        for q in range(NQ):
            pa = pa + jnp.abs(phq[q] * ih + prq[q] * ir - ptq[q] * it)
            na = na + jnp.abs(nhq[q] * jh + nrq[q] * jr - ntq[q] * jt)
        p = jnp.sum(pa)
        n = jnp.sum(na)
        pv = jnp.full((L,), p, jnp.float32)
        nv = jnp.full((L,), n, jnp.float32)
        plsc.store_scatter(pred_s, [jnp.full((L,), i, jnp.int32)],
                           pv, mask=lane0)
        return l_acc + jnp.maximum(pv - nv + MARGIN, 0.0)

    loss_acc = lax.fori_loop(0, BPW, row_body, zf)

    loss_s[...] = loss_acc * 0.0625
    pltpu.sync_copy(pred_s, pred_out.at[pl.ds(base, BPW)])
    pltpu.sync_copy(loss_s, loss_out.at[wid])


def kernel(pos_h, pos_t, pos_r, neg_h, neg_t, neg_r,
           ent_embeddings, rel_embeddings):
    mesh = plsc.VectorSubcoreMesh(core_axis_name="c", subcore_axis_name="s")
    run = pl.kernel(
        _body,
        out_type=(
            jax.ShapeDtypeStruct((B,), jnp.float32),
            jax.ShapeDtypeStruct((NW, L), jnp.float32),
        ),
        mesh=mesh,
        compiler_params=pltpu.CompilerParams(needs_layout_passes=False,
                                             use_tc_tiling_on_sc=True),
        scratch_types=(
            [pltpu.VMEM((BPW + L,), jnp.int32) for _ in range(6)]
            + [pltpu.VMEM((R * NT, D), jnp.float32),
               pltpu.VMEM((BPW,), jnp.float32),
               pltpu.VMEM((L,), jnp.float32),
               pltpu.SemaphoreType.DMA]
        ),
    )
    pred, loss_part = run(
        pos_h.astype(jnp.int32), pos_t.astype(jnp.int32),
        pos_r.astype(jnp.int32), neg_h.astype(jnp.int32),
        neg_t.astype(jnp.int32), neg_r.astype(jnp.int32),
        ent_embeddings, rel_embeddings)
    return (jnp.sum(loss_part), pred)


# speedup vs baseline: 1.0696x; 1.0028x over previous
import jax
import jax.numpy as jnp
from jax import lax
from jax.experimental import pallas as pl
from jax.experimental.pallas import tpu as pltpu
from jax.experimental.pallas import tpu_sc as plsc

D = 64
B = 16384
L = 16
NC, NS = 2, 16
NW = NC * NS
BPW = B // NW
R = 16
NT = 6
MARGIN = 1.0


def _rsqrt16(x):
    x = jnp.maximum(x, 1e-12)
    i = plsc.bitcast(x, jnp.int32)
    y = plsc.bitcast(jnp.full((L,), 0x5F3759DF, jnp.int32) - (i >> 1),
                     jnp.float32)
    for _ in range(3):
        y = y * (1.5 - 0.5 * x * y * y)
    return y


def _body(ph_i, pt_i, pr_i, nh_i, nt_i, nr_i, ent, rel,
          pred_out, loss_out,
          ph_x, pt_x, pr_x, nh_x, nt_x, nr_x,
          ring, pred_s, loss_s, sem):
    wid = lax.axis_index("s") * NC + lax.axis_index("c")
    base = wid * BPW
    row_iota = lax.iota(jnp.int32, L)
    lane0 = row_iota == 0
    zf = jnp.zeros((L,), jnp.float32)

    idx_refs = (ph_x, pt_x, pr_x, nh_x, nt_x, nr_x)
    idx_srcs = (ph_i, pt_i, pr_i, nh_i, nt_i, nr_i)
    tables = (ent, ent, rel, ent, ent, rel)
    NQ = D // L

    for src, dst in zip(idx_srcs, idx_refs):
        pltpu.sync_copy(src.at[pl.ds(base, BPW)], dst.at[pl.ds(0, BPW)])

    def fire(row):
        slot = lax.rem(row, R)
        for t, (tab, ix) in enumerate(zip(tables, idx_refs)):
            r0 = ix[pl.ds(row, L)][0]
            pltpu.async_copy(tab.at[pl.ds(r0, 1)],
                             ring.at[pl.ds(slot * NT + t, 1)], sem)

    def drain_one():
        pltpu.make_async_copy(ent.at[pl.ds(0, 1)],
                              ring.at[pl.ds(0, 1)], sem).wait()

    for j in range(R):
        fire(j)

    def row_body(i, l_acc):
        slot = lax.rem(i, R)
        for _ in range(NT):
            drain_one()
        quads = [[ring[slot * NT + t, pl.ds(L * q, L)] for q in range(NQ)]
                 for t in range(NT)]

        @pl.when(i < BPW - R)
        def _():
            fire(i + R)

        phq, ptq, prq, nhq, ntq, nrq = quads

        def inv_norm(vq):
            s = vq[0] * vq[0] + vq[1] * vq[1]
            s = s + vq[2] * vq[2] + vq[3] * vq[3]
            return _rsqrt16(jnp.full((L,), jnp.sum(s), jnp.float32))

        ih, it, ir, jh, jt, jr = [inv_norm(vq) for vq in quads]

        pa, na = zf, zf
        for q in range(NQ):
            pa = pa + jnp.abs(phq[q] * ih + prq[q] * ir - ptq[q] * it)
            na = na + jnp.abs(nhq[q] * jh + nrq[q] * jr - ntq[q] * jt)
        p = jnp.sum(pa)
        n = jnp.sum(na)
        pv = jnp.full((L,), p, jnp.float32)
        nv = jnp.full((L,), n, jnp.float32)
        plsc.store_scatter(pred_s, [jnp.full((L,), i, jnp.int32)],
                           pv, mask=lane0)
        return l_acc + jnp.maximum(pv - nv + MARGIN, 0.0)

    loss_acc = lax.fori_loop(0, BPW, row_body, zf)

    loss_s[...] = loss_acc * 0.0625
    pltpu.sync_copy(pred_s, pred_out.at[pl.ds(base, BPW)])
    pltpu.sync_copy(loss_s, loss_out.at[wid])


def kernel(pos_h, pos_t, pos_r, neg_h, neg_t, neg_r,
           ent_embeddings, rel_embeddings):
    mesh = plsc.VectorSubcoreMesh(core_axis_name="c", subcore_axis_name="s")
    run = pl.kernel(
        _body,
        out_type=(
            jax.ShapeDtypeStruct((B,), jnp.float32),
            jax.ShapeDtypeStruct((NW, L), jnp.float32),
        ),
        mesh=mesh,
        compiler_params=pltpu.CompilerParams(needs_layout_passes=False,
                                             use_tc_tiling_on_sc=True),
        scratch_types=(
            [pltpu.VMEM((BPW + L,), jnp.int32) for _ in range(6)]
            + [pltpu.VMEM((R * NT, D), jnp.float32),
               pltpu.VMEM((BPW,), jnp.float32),
               pltpu.VMEM((L,), jnp.float32),
               pltpu.SemaphoreType.DMA]
        ),
    )
    pred, loss_part = run(
        pos_h.astype(jnp.int32), pos_t.astype(jnp.int32),
        pos_r.astype(jnp.int32), neg_h.astype(jnp.int32),
        neg_t.astype(jnp.int32), neg_r.astype(jnp.int32),
        ent_embeddings, rel_embeddings)
    return (jnp.sum(loss_part), pred)


# speedup vs baseline: 1.0943x; 1.0231x over previous
import jax
import jax.numpy as jnp
from jax import lax
from jax.experimental import pallas as pl
from jax.experimental.pallas import tpu as pltpu
from jax.experimental.pallas import tpu_sc as plsc

D = 64
B = 16384
L = 16
NC, NS = 2, 16
NW = NC * NS
BPW = B // NW
R = 16
NT = 4
MARGIN = 1.0


def _rsqrt16(x):
    x = jnp.maximum(x, 1e-12)
    i = plsc.bitcast(x, jnp.int32)
    y = plsc.bitcast(jnp.full((L,), 0x5F3759DF, jnp.int32) - (i >> 1),
                     jnp.float32)
    for _ in range(3):
        y = y * (1.5 - 0.5 * x * y * y)
    return y


def _body(ph_i, pt_i, pr_i, nh_i, nt_i, nr_i, ent, rel,
          pred_out, loss_out,
          ph_x, pt_x, pr_x, nh_x, nt_x, nr_x,
          ring, rel_v, pred_s, loss_s, sem):
    wid = lax.axis_index("s") * NC + lax.axis_index("c")
    base = wid * BPW
    row_iota = lax.iota(jnp.int32, L)
    lane0 = row_iota == 0
    zf = jnp.zeros((L,), jnp.float32)

    idx_refs = (ph_x, pt_x, nh_x, nt_x)
    rel_idx = (pr_x, nr_x)
    idx_srcs = (ph_i, pt_i, nh_i, nt_i)
    rel_srcs = (pr_i, nr_i)
    NQ = D // L

    for src, dst in zip(idx_srcs + rel_srcs, idx_refs + rel_idx):
        pltpu.sync_copy(src.at[pl.ds(base, BPW)], dst.at[pl.ds(0, BPW)])
    pltpu.sync_copy(rel, rel_v)

    def fire(row):
        slot = lax.rem(row, R)
        for t, ix in enumerate(idx_refs):
            r0 = ix[pl.ds(row, L)][0]
            pltpu.async_copy(ent.at[pl.ds(r0, 1)],
                             ring.at[pl.ds(slot * NT + t, 1)], sem)

    def drain_one():
        pltpu.make_async_copy(ent.at[pl.ds(0, 1)],
                              ring.at[pl.ds(0, 1)], sem).wait()

    for j in range(R):
        fire(j)

    def row_body(i, l_acc):
        slot = lax.rem(i, R)
        for _ in range(NT):
            drain_one()
        quads = [[ring[slot * NT + t, pl.ds(L * q, L)] for q in range(NQ)]
                 for t in range(NT)]
        for ix in rel_idx:
            ri = ix[pl.ds(i, L)][0]
            off = (ri & 1) * D
            quads.append([rel_v[ri >> 1, pl.ds(off + L * q, L)]
                          for q in range(NQ)])

        @pl.when(i < BPW - R)
        def _():
            fire(i + R)

        phq, ptq, nhq, ntq, prq, nrq = quads

        def inv_norm(vq):
            s = vq[0] * vq[0] + vq[1] * vq[1]
            s = s + vq[2] * vq[2] + vq[3] * vq[3]
            return _rsqrt16(jnp.full((L,), jnp.sum(s), jnp.float32))

        ih, it, jh, jt, ir, jr = [inv_norm(vq) for vq in quads]

        pa, na = zf, zf
        for q in range(NQ):
            pa = pa + jnp.abs(phq[q] * ih + prq[q] * ir - ptq[q] * it)
            na = na + jnp.abs(nhq[q] * jh + nrq[q] * jr - ntq[q] * jt)
        p = jnp.sum(pa)
        n = jnp.sum(na)
        pv = jnp.full((L,), p, jnp.float32)
        nv = jnp.full((L,), n, jnp.float32)
        plsc.store_scatter(pred_s, [jnp.full((L,), i, jnp.int32)],
                           pv, mask=lane0)
        return l_acc + jnp.maximum(pv - nv + MARGIN, 0.0)

    loss_acc = lax.fori_loop(0, BPW, row_body, zf)

    loss_s[...] = loss_acc * 0.0625
    pltpu.sync_copy(pred_s, pred_out.at[pl.ds(base, BPW)])
    pltpu.sync_copy(loss_s, loss_out.at[wid])


def kernel(pos_h, pos_t, pos_r, neg_h, neg_t, neg_r,
           ent_embeddings, rel_embeddings):
    mesh = plsc.VectorSubcoreMesh(core_axis_name="c", subcore_axis_name="s")
    run = pl.kernel(
        _body,
        out_type=(
            jax.ShapeDtypeStruct((B,), jnp.float32),
            jax.ShapeDtypeStruct((NW, L), jnp.float32),
        ),
        mesh=mesh,
        compiler_params=pltpu.CompilerParams(needs_layout_passes=False,
                                             use_tc_tiling_on_sc=True),
        scratch_types=(
            [pltpu.VMEM((BPW + L,), jnp.int32) for _ in range(6)]
            + [pltpu.VMEM((R * NT, D), jnp.float32),
               pltpu.VMEM((500, 2 * D), jnp.float32),
               pltpu.VMEM((BPW,), jnp.float32),
               pltpu.VMEM((L,), jnp.float32),
               pltpu.SemaphoreType.DMA]
        ),
    )
    pred, loss_part = run(
        pos_h.astype(jnp.int32), pos_t.astype(jnp.int32),
        pos_r.astype(jnp.int32), neg_h.astype(jnp.int32),
        neg_t.astype(jnp.int32), neg_r.astype(jnp.int32),
        ent_embeddings,
        rel_embeddings.reshape(rel_embeddings.shape[0] // 2, 2 * D))
    return (jnp.sum(loss_part), pred)


# speedup vs baseline: 1.1042x; 1.0091x over previous
import jax
import jax.numpy as jnp
from jax import lax
from jax.experimental import pallas as pl
from jax.experimental.pallas import tpu as pltpu
from jax.experimental.pallas import tpu_sc as plsc

D = 64
B = 16384
L = 16
NC, NS = 2, 16
NW = NC * NS
BPW = B // NW
R = 16
NT = 4
MARGIN = 1.0


def _rsqrt16(x):
    x = jnp.maximum(x, 1e-12)
    i = plsc.bitcast(x, jnp.int32)
    y = plsc.bitcast(jnp.full((L,), 0x5F3759DF, jnp.int32) - (i >> 1),
                     jnp.float32)
    for _ in range(3):
        y = y * (1.5 - 0.5 * x * y * y)
    return y


def _body(ph_i, pt_i, pr_i, nh_i, nt_i, nr_i, ent, rel,
          pred_out, loss_out,
          ph_x, pt_x, pr_x, nh_x, nt_x, nr_x,
          ring, rel_v, pred_s, loss_s, sem):
    wid = lax.axis_index("s") * NC + lax.axis_index("c")
    base = wid * BPW
    row_iota = lax.iota(jnp.int32, L)
    lane0 = row_iota == 0
    zf = jnp.zeros((L,), jnp.float32)

    idx_refs = (ph_x, pt_x, nh_x, nt_x)
    rel_idx = (pr_x, nr_x)
    idx_srcs = (ph_i, pt_i, nh_i, nt_i)
    rel_srcs = (pr_i, nr_i)
    NQ = D // L

    for src, dst in zip(idx_srcs + rel_srcs, idx_refs + rel_idx):
        pltpu.sync_copy(src.at[pl.ds(base, BPW)], dst.at[pl.ds(0, BPW)])
    pltpu.sync_copy(rel, rel_v)

    def fire(row):
        slot = lax.rem(row, R)
        for t, ix in enumerate(idx_refs):
            r0 = ix[pl.ds(row, L)][0]
            pltpu.async_copy(ent.at[pl.ds(r0, 1)],
                             ring.at[pl.ds(slot * NT + t, 1)], sem)

    def drain_one():
        pltpu.make_async_copy(ent.at[pl.ds(0, 1)],
                              ring.at[pl.ds(0, 1)], sem).wait()

    for j in range(R):
        fire(j)

    def row_body(i, l_acc):
        slot = lax.rem(i, R)

        @pl.when((i & 1) == 0)
        def _():
            pltpu.make_async_copy(ent.at[pl.ds(0, 8)],
                                  ring.at[pl.ds(slot * NT, 8)], sem).wait()
        quads = [[ring[slot * NT + t, pl.ds(L * q, L)] for q in range(NQ)]
                 for t in range(NT)]
        for ix in rel_idx:
            ri = ix[pl.ds(i, L)][0]
            off = (ri & 1) * D
            quads.append([rel_v[ri >> 1, pl.ds(off + L * q, L)]
                          for q in range(NQ)])

        @pl.when(i < BPW - R)
        def _():
            fire(i + R)

        phq, ptq, nhq, ntq, prq, nrq = quads

        def inv_norm(vq):
            s = vq[0] * vq[0] + vq[1] * vq[1]
            s = s + vq[2] * vq[2] + vq[3] * vq[3]
            return _rsqrt16(jnp.full((L,), jnp.sum(s), jnp.float32))

        ih, it, jh, jt, ir, jr = [inv_norm(vq) for vq in quads]

        pa, na = zf, zf
        for q in range(NQ):
            pa = pa + jnp.abs(phq[q] * ih + prq[q] * ir - ptq[q] * it)
            na = na + jnp.abs(nhq[q] * jh + nrq[q] * jr - ntq[q] * jt)
        p = jnp.sum(pa)
        n = jnp.sum(na)
        pv = jnp.full((L,), p, jnp.float32)
        nv = jnp.full((L,), n, jnp.float32)
        plsc.store_scatter(pred_s, [jnp.full((L,), i, jnp.int32)],
                           pv, mask=lane0)
        return l_acc + jnp.maximum(pv - nv + MARGIN, 0.0)

    loss_acc = lax.fori_loop(0, BPW, row_body, zf)

    loss_s[...] = loss_acc * 0.0625
    pltpu.sync_copy(pred_s, pred_out.at[pl.ds(base, BPW)])
    pltpu.sync_copy(loss_s, loss_out.at[wid])


def kernel(pos_h, pos_t, pos_r, neg_h, neg_t, neg_r,
           ent_embeddings, rel_embeddings):
    mesh = plsc.VectorSubcoreMesh(core_axis_name="c", subcore_axis_name="s")
    run = pl.kernel(
        _body,
        out_type=(
            jax.ShapeDtypeStruct((B,), jnp.float32),
            jax.ShapeDtypeStruct((NW, L), jnp.float32),
        ),
        mesh=mesh,
        compiler_params=pltpu.CompilerParams(needs_layout_passes=False,
                                             use_tc_tiling_on_sc=True),
        scratch_types=(
            [pltpu.VMEM((BPW + L,), jnp.int32) for _ in range(6)]
            + [pltpu.VMEM((R * NT, D), jnp.float32),
               pltpu.VMEM((500, 2 * D), jnp.float32),
               pltpu.VMEM((BPW,), jnp.float32),
               pltpu.VMEM((L,), jnp.float32),
               pltpu.SemaphoreType.DMA]
        ),
    )
    pred, loss_part = run(
        pos_h.astype(jnp.int32), pos_t.astype(jnp.int32),
        pos_r.astype(jnp.int32), neg_h.astype(jnp.int32),
        neg_t.astype(jnp.int32), neg_r.astype(jnp.int32),
        ent_embeddings,
        rel_embeddings.reshape(rel_embeddings.shape[0] // 2, 2 * D))
    return (jnp.sum(loss_part), pred)
